# Initial kernel scaffold; baseline (speedup 1.0000x reference)
import functools

import jax
import jax.numpy as jnp
from jax import lax
from jax.experimental import pallas as pl
from jax.experimental.pallas import tpu as pltpu
from jax.experimental.pallas import tpu_sc as plsc


def _mm_body(x_ref, w_ref, b_ref, o_ref):
    o_ref[...] = (
        jnp.dot(x_ref[...], w_ref[...], preferred_element_type=jnp.float32)
        + b_ref[...]
    )


def _matmul(x, w, b, bm):
    m, kdim = x.shape
    n = w.shape[1]
    return pl.pallas_call(
        _mm_body,
        grid=(m // bm,),
        in_specs=[
            pl.BlockSpec((bm, kdim), lambda i: (i, 0)),
            pl.BlockSpec((kdim, n), lambda i: (0, 0)),
            pl.BlockSpec((1, n), lambda i: (0, 0)),
        ],
        out_specs=pl.BlockSpec((bm, n), lambda i: (i, 0)),
        out_shape=jax.ShapeDtypeStruct((m, n), jnp.float32),
    )(x, w, b.reshape(1, n))


def _qkvs_body(h_ref, wk_ref, wq_ref, wv_ref, ws_ref,
               bk_ref, bq_ref, bv_ref, bs_ref,
               k_ref, qv_ref, s_ref):
    h = h_ref[...]
    hd = wk_ref.shape[1]
    k_ref[...] = jnp.dot(h, wk_ref[...], preferred_element_type=jnp.float32) + bk_ref[...]
    qv_ref[:, :hd] = jnp.dot(h, wq_ref[...], preferred_element_type=jnp.float32) + bq_ref[...]
    qv_ref[:, hd:] = jnp.dot(h, wv_ref[...], preferred_element_type=jnp.float32) + bv_ref[...]
    s_ref[...] = jnp.dot(h, ws_ref[...], preferred_element_type=jnp.float32) + bs_ref[...]


def _qkvs(h, wk, wq, wv, ws, bk, bq, bv, bs, bm):
    m, hd = h.shape
    wspec = pl.BlockSpec((hd, hd), lambda i: (0, 0))
    bspec = pl.BlockSpec((1, hd), lambda i: (0, 0))
    return pl.pallas_call(
        _qkvs_body,
        grid=(m // bm,),
        in_specs=[pl.BlockSpec((bm, hd), lambda i: (i, 0))]
        + [wspec] * 4 + [bspec] * 4,
        out_specs=[
            pl.BlockSpec((bm, hd), lambda i: (i, 0)),
            pl.BlockSpec((bm, 2 * hd), lambda i: (i, 0)),
            pl.BlockSpec((bm, hd), lambda i: (i, 0)),
        ],
        out_shape=[
            jax.ShapeDtypeStruct((m, hd), jnp.float32),
            jax.ShapeDtypeStruct((m, 2 * hd), jnp.float32),
            jax.ShapeDtypeStruct((m, hd), jnp.float32),
        ],
    )(h, wk, wq, wv, ws,
      bk.reshape(1, hd), bq.reshape(1, hd), bv.reshape(1, hd), bs.reshape(1, hd))


def _ee_body(ea_ref, we_ref, wed_ref, be_ref, bed_ref, o_ref):
    u = jnp.dot(we_ref[...], wed_ref[...], preferred_element_type=jnp.float32)
    c = jnp.dot(be_ref[...], wed_ref[...], preferred_element_type=jnp.float32) + bed_ref[...]
    o_ref[...] = jnp.dot(ea_ref[...], u, preferred_element_type=jnp.float32) + c


def _ee(edge_attr, we, wed_i, be, bed_i, be_blk):
    e_cnt, de = edge_attr.shape
    hd = we.shape[1]
    return pl.pallas_call(
        _ee_body,
        grid=(e_cnt // be_blk,),
        in_specs=[
            pl.BlockSpec((be_blk, de), lambda i: (i, 0)),
            pl.BlockSpec((de, hd), lambda i: (0, 0)),
            pl.BlockSpec((hd, hd), lambda i: (0, 0)),
            pl.BlockSpec((1, hd), lambda i: (0, 0)),
            pl.BlockSpec((1, hd), lambda i: (0, 0)),
        ],
        out_specs=pl.BlockSpec((be_blk, hd), lambda i: (i, 0)),
        out_shape=jax.ShapeDtypeStruct((e_cnt, hd), jnp.float32),
    )(edge_attr, we, wed_i, be.reshape(1, hd), bed_i.reshape(1, hd))


def _bn_update(agg2, s, h, gamma_i, beta_i):
    n_nodes, hd = h.shape

    def body(agg_ref, s_ref, h_ref, g_ref, b_ref, o_ref):
        a = agg_ref[...]
        nnew = a[:n_nodes] + a[n_nodes:] + s_ref[...]
        mean = jnp.mean(nnew, axis=0, keepdims=True)
        ctr = nnew - mean
        var = jnp.mean(ctr * ctr, axis=0, keepdims=True)
        nb = g_ref[...] * ctr * lax.rsqrt(var + 1e-5) + b_ref[...]
        o_ref[...] = (h_ref[...] + jnp.maximum(nb, 0.0)) * 0.5

    return pl.pallas_call(
        body,
        out_shape=jax.ShapeDtypeStruct((n_nodes, hd), jnp.float32),
    )(agg2, s, h, gamma_i.reshape(1, hd), beta_i.reshape(1, hd))


def _make_edge_pass(n_nodes, n_edges, hd):
    C = 128
    nchunks = n_edges // C
    NC, NS = 2, 16
    NW = NC * NS
    max_it = (nchunks + NW - 1) // NW
    rows_per_tile = n_nodes // NS
    RB = 125
    n_rb = rows_per_tile // RB
    nslice = hd // 16

    mesh = plsc.VectorSubcoreMesh(core_axis_name="c", subcore_axis_name="s")

    @functools.partial(
        pl.kernel,
        mesh=mesh,
        out_type=jax.ShapeDtypeStruct((NC * n_nodes, hd), jnp.float32),
        scratch_types=[
            pltpu.VMEM((C,), jnp.int32),
            pltpu.VMEM((C,), jnp.int32),
            pltpu.VMEM((C, hd), jnp.float32),
            pltpu.VMEM((C, 2 * hd), jnp.float32),
            pltpu.VMEM((C, hd), jnp.float32),
            pltpu.VMEM((RB, hd), jnp.float32),
            pltpu.VMEM_SHARED((n_nodes, hd), jnp.float32),
        ],
    )
    def edge_pass(k_hbm, qv_hbm, ee_hbm, src_hbm, dst_hbm, out_hbm,
                  sidx, didx, kbuf, qvbuf, ebuf, zbuf, acc):
        cid = lax.axis_index("c")
        sid = lax.axis_index("s")
        wid = sid * NC + cid

        def zrow(r, carry):
            for j in range(nslice):
                zbuf[r, pl.ds(j * 16, 16)] = jnp.zeros((16,), jnp.float32)
            return carry

        lax.fori_loop(0, RB, zrow, 0)
        for t in range(n_rb):
            pltpu.sync_copy(zbuf, acc.at[pl.ds(sid * rows_per_tile + t * RB, RB)])
        plsc.subcore_barrier()

        def chunk_body(it, carry):
            ch = it * NW + wid

            @pl.when(ch < nchunks)
            def _():
                base = ch * C
                pltpu.sync_copy(src_hbm.at[pl.ds(base, C)], sidx)
                pltpu.sync_copy(dst_hbm.at[pl.ds(base, C)], didx)
                pltpu.sync_copy(k_hbm.at[didx], kbuf)
                pltpu.sync_copy(qv_hbm.at[sidx], qvbuf)
                pltpu.sync_copy(ee_hbm.at[pl.ds(base, C)], ebuf)

                def row(r, c2):
                    for j in range(nslice):
                        sl = pl.ds(j * 16, 16)
                        g = kbuf[r, sl] + qvbuf[r, sl] + ebuf[r, sl]
                        g = 1.0 / (1.0 + jnp.exp(-g))
                        ebuf[r, sl] = g * qvbuf[r, pl.ds(hd + j * 16, 16)]
                    return c2

                lax.fori_loop(0, C, row, 0)
                pltpu.sync_copy(ebuf, acc.at[didx], add=True)

            return carry

        lax.fori_loop(0, max_it, chunk_body, 0)
        plsc.subcore_barrier()

        for t in range(n_rb):
            r0 = sid * rows_per_tile + t * RB
            pltpu.sync_copy(acc.at[pl.ds(r0, RB)], zbuf)
            pltpu.sync_copy(zbuf, out_hbm.at[pl.ds(cid * n_nodes + r0, RB)])

    return edge_pass


def kernel(x, edge_index, edge_attr, Wn, bn_, We, be, Wk, bk, Wq, bq,
           Wv, bv, Ws, bs, Wed, bed, gamma, beta, Wh, bh):
    n_nodes = x.shape[0]
    n_edges = edge_index.shape[1]
    hd = Wn.shape[1]
    n_layers = Wk.shape[0]

    src = edge_index[0]
    dst = edge_index[1]

    h = _matmul(x, Wn, bn_, 2000)
    edge_pass = _make_edge_pass(n_nodes, n_edges, hd)

    for i in range(n_layers):
        ee = _ee(edge_attr, We, Wed[i], be, bed[i], 8000)
        k, qv, s = _qkvs(h, Wk[i], Wq[i], Wv[i], Ws[i],
                         bk[i], bq[i], bv[i], bs[i], 2000)
        agg2 = edge_pass(k, qv, ee, src, dst)
        h = _bn_update(agg2, s, h, gamma[i], beta[i])

    return _matmul(h, Wh, bh, 2000)


# speedup vs baseline: 1.1883x; 1.1883x over previous
import functools

import jax
import jax.numpy as jnp
from jax import lax
from jax.experimental import pallas as pl
from jax.experimental.pallas import tpu as pltpu
from jax.experimental.pallas import tpu_sc as plsc


def _mm_body(x_ref, w_ref, b_ref, o_ref):
    o_ref[...] = (
        jnp.dot(x_ref[...], w_ref[...], preferred_element_type=jnp.float32)
        + b_ref[...]
    )


def _matmul(x, w, b, bm):
    m, kdim = x.shape
    n = w.shape[1]
    return pl.pallas_call(
        _mm_body,
        grid=(m // bm,),
        in_specs=[
            pl.BlockSpec((bm, kdim), lambda i: (i, 0)),
            pl.BlockSpec((kdim, n), lambda i: (0, 0)),
            pl.BlockSpec((1, n), lambda i: (0, 0)),
        ],
        out_specs=pl.BlockSpec((bm, n), lambda i: (i, 0)),
        out_shape=jax.ShapeDtypeStruct((m, n), jnp.float32),
    )(x, w, b.reshape(1, n))


def _qkvs_body(h_ref, wk_ref, wq_ref, wv_ref, ws_ref,
               bk_ref, bq_ref, bv_ref, bs_ref,
               k_ref, qv_ref, s_ref):
    h = h_ref[...]
    hd = wk_ref.shape[1]
    k_ref[...] = jnp.dot(h, wk_ref[...], preferred_element_type=jnp.float32) + bk_ref[...]
    qv_ref[:, :hd] = jnp.dot(h, wq_ref[...], preferred_element_type=jnp.float32) + bq_ref[...]
    qv_ref[:, hd:] = jnp.dot(h, wv_ref[...], preferred_element_type=jnp.float32) + bv_ref[...]
    s_ref[...] = jnp.dot(h, ws_ref[...], preferred_element_type=jnp.float32) + bs_ref[...]


def _qkvs(h, wk, wq, wv, ws, bk, bq, bv, bs, bm):
    m, hd = h.shape
    wspec = pl.BlockSpec((hd, hd), lambda i: (0, 0))
    bspec = pl.BlockSpec((1, hd), lambda i: (0, 0))
    return pl.pallas_call(
        _qkvs_body,
        grid=(m // bm,),
        in_specs=[pl.BlockSpec((bm, hd), lambda i: (i, 0))]
        + [wspec] * 4 + [bspec] * 4,
        out_specs=[
            pl.BlockSpec((bm, hd), lambda i: (i, 0)),
            pl.BlockSpec((bm, 2 * hd), lambda i: (i, 0)),
            pl.BlockSpec((bm, hd), lambda i: (i, 0)),
        ],
        out_shape=[
            jax.ShapeDtypeStruct((m, hd), jnp.float32),
            jax.ShapeDtypeStruct((m, 2 * hd), jnp.float32),
            jax.ShapeDtypeStruct((m, hd), jnp.float32),
        ],
    )(h, wk, wq, wv, ws,
      bk.reshape(1, hd), bq.reshape(1, hd), bv.reshape(1, hd), bs.reshape(1, hd))


def _ee_body(ea_ref, we_ref, wed_ref, be_ref, bed_ref, o_ref):
    u = jnp.dot(we_ref[...], wed_ref[...], preferred_element_type=jnp.float32)
    c = jnp.dot(be_ref[...], wed_ref[...], preferred_element_type=jnp.float32) + bed_ref[...]
    o_ref[...] = jnp.dot(ea_ref[...], u, preferred_element_type=jnp.float32) + c


def _ee(edge_attr, we, wed_i, be, bed_i, be_blk):
    e_cnt, de = edge_attr.shape
    hd = we.shape[1]
    return pl.pallas_call(
        _ee_body,
        grid=(e_cnt // be_blk,),
        in_specs=[
            pl.BlockSpec((be_blk, de), lambda i: (i, 0)),
            pl.BlockSpec((de, hd), lambda i: (0, 0)),
            pl.BlockSpec((hd, hd), lambda i: (0, 0)),
            pl.BlockSpec((1, hd), lambda i: (0, 0)),
            pl.BlockSpec((1, hd), lambda i: (0, 0)),
        ],
        out_specs=pl.BlockSpec((be_blk, hd), lambda i: (i, 0)),
        out_shape=jax.ShapeDtypeStruct((e_cnt, hd), jnp.float32),
    )(edge_attr, we, wed_i, be.reshape(1, hd), bed_i.reshape(1, hd))


def _bn_update(agg2, s, h, gamma_i, beta_i):
    n_nodes, hd = h.shape

    def body(agg_ref, s_ref, h_ref, g_ref, b_ref, o_ref):
        a = agg_ref[...]
        nnew = a[:n_nodes] + a[n_nodes:] + s_ref[...]
        mean = jnp.mean(nnew, axis=0, keepdims=True)
        ctr = nnew - mean
        var = jnp.mean(ctr * ctr, axis=0, keepdims=True)
        nb = g_ref[...] * ctr * lax.rsqrt(var + 1e-5) + b_ref[...]
        o_ref[...] = (h_ref[...] + jnp.maximum(nb, 0.0)) * 0.5

    return pl.pallas_call(
        body,
        out_shape=jax.ShapeDtypeStruct((n_nodes, hd), jnp.float32),
    )(agg2, s, h, gamma_i.reshape(1, hd), beta_i.reshape(1, hd))


def _make_edge_pass(n_nodes, n_edges, hd):
    C = 64
    nchunks = n_edges // C
    NC, NS = 2, 16
    NW = NC * NS
    max_it = (nchunks + NW - 1) // NW
    RT = (n_nodes // NS) // 8 * 8
    RB = 48
    n_rb = RT // RB
    tail = n_nodes - NS * RT
    nslice = hd // 16

    mesh = plsc.VectorSubcoreMesh(core_axis_name="c", subcore_axis_name="s")

    @functools.partial(
        pl.kernel,
        mesh=mesh,
        out_type=jax.ShapeDtypeStruct((NC * n_nodes, hd), jnp.float32),
        scratch_types=[
            pltpu.VMEM((C,), jnp.int32),
            pltpu.VMEM((C,), jnp.int32),
            pltpu.VMEM((C, hd), jnp.float32),
            pltpu.VMEM((C, 2 * hd), jnp.float32),
            pltpu.VMEM((C, hd), jnp.float32),
            pltpu.VMEM((RB, hd), jnp.float32),
            pltpu.VMEM_SHARED((n_nodes, hd), jnp.float32),
        ],
    )
    def edge_pass(k_hbm, qv_hbm, ee_hbm, src_hbm, dst_hbm, out_hbm,
                  sidx, didx, kbuf, qvbuf, ebuf, zbuf, acc):
        cid = lax.axis_index("c")
        sid = lax.axis_index("s")
        wid = sid * NC + cid

        def zrow(r, carry):
            for j in range(nslice):
                zbuf[r, pl.ds(j * 16, 16)] = jnp.zeros((16,), jnp.float32)
            return carry

        lax.fori_loop(0, RB, zrow, 0)
        for t in range(n_rb):
            pltpu.sync_copy(zbuf, acc.at[pl.ds(sid * RT + t * RB, RB)])

        @pl.when(sid == 0)
        def _zero_tail():
            pltpu.sync_copy(zbuf.at[pl.ds(0, tail)], acc.at[pl.ds(NS * RT, tail)])

        plsc.subcore_barrier()

        def chunk_body(it, carry):
            ch = it * NW + wid

            @pl.when(ch < nchunks)
            def _():
                base = ch * C
                pltpu.sync_copy(src_hbm.at[pl.ds(base, C)], sidx)
                pltpu.sync_copy(dst_hbm.at[pl.ds(base, C)], didx)
                pltpu.sync_copy(k_hbm.at[didx], kbuf)
                pltpu.sync_copy(qv_hbm.at[sidx], qvbuf)
                pltpu.sync_copy(ee_hbm.at[pl.ds(base, C)], ebuf)

                def row(r, c2):
                    for j in range(nslice):
                        sl = pl.ds(j * 16, 16)
                        g = kbuf[r, sl] + qvbuf[r, sl] + ebuf[r, sl]
                        g = 1.0 / (1.0 + jnp.exp(-g))
                        ebuf[r, sl] = g * qvbuf[r, pl.ds(hd + j * 16, 16)]
                    return c2

                lax.fori_loop(0, C, row, 0)
                pltpu.sync_copy(ebuf, acc.at[didx], add=True)

            return carry

        lax.fori_loop(0, max_it, chunk_body, 0)
        plsc.subcore_barrier()

        for t in range(n_rb):
            r0 = sid * RT + t * RB
            pltpu.sync_copy(acc.at[pl.ds(r0, RB)], zbuf)
            pltpu.sync_copy(zbuf, out_hbm.at[pl.ds(cid * n_nodes + r0, RB)])

        @pl.when(sid == 0)
        def _copy_tail():
            pltpu.sync_copy(acc.at[pl.ds(NS * RT, tail)], zbuf.at[pl.ds(0, tail)])
            pltpu.sync_copy(zbuf.at[pl.ds(0, tail)],
                            out_hbm.at[pl.ds(cid * n_nodes + NS * RT, tail)])

    return edge_pass


def kernel(x, edge_index, edge_attr, Wn, bn_, We, be, Wk, bk, Wq, bq,
           Wv, bv, Ws, bs, Wed, bed, gamma, beta, Wh, bh):
    n_nodes = x.shape[0]
    n_edges = edge_index.shape[1]
    hd = Wn.shape[1]
    n_layers = Wk.shape[0]

    src = edge_index[0]
    dst = edge_index[1]

    h = _matmul(x, Wn, bn_, 2000)
    edge_pass = _make_edge_pass(n_nodes, n_edges, hd)

    for i in range(n_layers):
        ee = _ee(edge_attr, We, Wed[i], be, bed[i], 8000)
        k, qv, s = _qkvs(h, Wk[i], Wq[i], Wv[i], Ws[i],
                         bk[i], bq[i], bv[i], bs[i], 2000)
        agg2 = edge_pass(k, qv, ee, src, dst)
        h = _bn_update(agg2, s, h, gamma[i], beta[i])

    return _matmul(h, Wh, bh, 2000)


# speedup vs baseline: 2.9419x; 2.4756x over previous
import functools

import jax
import jax.numpy as jnp
from jax import lax
from jax.experimental import pallas as pl
from jax.experimental.pallas import tpu as pltpu
from jax.experimental.pallas import tpu_sc as plsc


def _mm_body(x_ref, w_ref, b_ref, o_ref):
    o_ref[...] = (
        jnp.dot(x_ref[...], w_ref[...], preferred_element_type=jnp.float32)
        + b_ref[...]
    )


def _matmul(x, w, b, bm):
    m, kdim = x.shape
    n = w.shape[1]
    return pl.pallas_call(
        _mm_body,
        grid=(m // bm,),
        in_specs=[
            pl.BlockSpec((bm, kdim), lambda i: (i, 0)),
            pl.BlockSpec((kdim, n), lambda i: (0, 0)),
            pl.BlockSpec((1, n), lambda i: (0, 0)),
        ],
        out_specs=pl.BlockSpec((bm, n), lambda i: (i, 0)),
        out_shape=jax.ShapeDtypeStruct((m, n), jnp.float32),
    )(x, w, b.reshape(1, n))


def _qkvs_body(h_ref, wk_ref, wq_ref, wv_ref, ws_ref,
               bk_ref, bq_ref, bv_ref, bs_ref,
               k_ref, qv_ref, s_ref):
    h = h_ref[...]
    hd = wk_ref.shape[1]
    k_ref[...] = jnp.dot(h, wk_ref[...], preferred_element_type=jnp.float32) + bk_ref[...]
    qv_ref[:, :hd] = jnp.dot(h, wq_ref[...], preferred_element_type=jnp.float32) + bq_ref[...]
    qv_ref[:, hd:] = jnp.dot(h, wv_ref[...], preferred_element_type=jnp.float32) + bv_ref[...]
    s_ref[...] = jnp.dot(h, ws_ref[...], preferred_element_type=jnp.float32) + bs_ref[...]


def _qkvs(h, wk, wq, wv, ws, bk, bq, bv, bs, bm):
    m, hd = h.shape
    wspec = pl.BlockSpec((hd, hd), lambda i: (0, 0))
    bspec = pl.BlockSpec((1, hd), lambda i: (0, 0))
    return pl.pallas_call(
        _qkvs_body,
        grid=(m // bm,),
        in_specs=[pl.BlockSpec((bm, hd), lambda i: (i, 0))]
        + [wspec] * 4 + [bspec] * 4,
        out_specs=[
            pl.BlockSpec((bm, hd), lambda i: (i, 0)),
            pl.BlockSpec((bm, 2 * hd), lambda i: (i, 0)),
            pl.BlockSpec((bm, hd), lambda i: (i, 0)),
        ],
        out_shape=[
            jax.ShapeDtypeStruct((m, hd), jnp.float32),
            jax.ShapeDtypeStruct((m, 2 * hd), jnp.float32),
            jax.ShapeDtypeStruct((m, hd), jnp.float32),
        ],
    )(h, wk, wq, wv, ws,
      bk.reshape(1, hd), bq.reshape(1, hd), bv.reshape(1, hd), bs.reshape(1, hd))


def _ee_body(ea_ref, we_ref, wed_ref, be_ref, bed_ref, o_ref):
    u = jnp.dot(we_ref[...], wed_ref[...], preferred_element_type=jnp.float32)
    c = jnp.dot(be_ref[...], wed_ref[...], preferred_element_type=jnp.float32) + bed_ref[...]
    o_ref[...] = jnp.dot(ea_ref[...], u, preferred_element_type=jnp.float32) + c


def _ee(edge_attr, we, wed_i, be, bed_i, be_blk):
    e_cnt, de = edge_attr.shape
    hd = we.shape[1]
    return pl.pallas_call(
        _ee_body,
        grid=(e_cnt // be_blk,),
        in_specs=[
            pl.BlockSpec((be_blk, de), lambda i: (i, 0)),
            pl.BlockSpec((de, hd), lambda i: (0, 0)),
            pl.BlockSpec((hd, hd), lambda i: (0, 0)),
            pl.BlockSpec((1, hd), lambda i: (0, 0)),
            pl.BlockSpec((1, hd), lambda i: (0, 0)),
        ],
        out_specs=pl.BlockSpec((be_blk, hd), lambda i: (i, 0)),
        out_shape=jax.ShapeDtypeStruct((e_cnt, hd), jnp.float32),
    )(edge_attr, we, wed_i, be.reshape(1, hd), bed_i.reshape(1, hd))


def _bn_update(agg2, s, h, gamma_i, beta_i):
    n_nodes, hd = h.shape

    def body(agg_ref, s_ref, h_ref, g_ref, b_ref, o_ref):
        a = agg_ref[...]
        nnew = a[:n_nodes] + a[n_nodes:] + s_ref[...]
        mean = jnp.mean(nnew, axis=0, keepdims=True)
        ctr = nnew - mean
        var = jnp.mean(ctr * ctr, axis=0, keepdims=True)
        nb = g_ref[...] * ctr * lax.rsqrt(var + 1e-5) + b_ref[...]
        o_ref[...] = (h_ref[...] + jnp.maximum(nb, 0.0)) * 0.5

    return pl.pallas_call(
        body,
        out_shape=jax.ShapeDtypeStruct((n_nodes, hd), jnp.float32),
    )(agg2, s, h, gamma_i.reshape(1, hd), beta_i.reshape(1, hd))


def _make_edge_pass(n_nodes, n_edges, hd):
    C = 64
    nchunks = n_edges // C
    NC, NS = 2, 16
    NW = NC * NS
    max_it = (nchunks + NW - 1) // NW
    RT = (n_nodes // NS) // 8 * 8
    RB = 48
    n_rb = RT // RB
    tail = n_nodes - NS * RT
    nslice = hd // 16

    mesh = plsc.VectorSubcoreMesh(core_axis_name="c", subcore_axis_name="s")

    @functools.partial(
        pl.kernel,
        mesh=mesh,
        out_type=jax.ShapeDtypeStruct((NC * n_nodes, hd), jnp.float32),
        scratch_types=[
            pltpu.VMEM((C,), jnp.int32),
            pltpu.VMEM((C,), jnp.int32),
            pltpu.VMEM((C, hd), jnp.float32),
            pltpu.VMEM((C, 2 * hd), jnp.float32),
            pltpu.VMEM((C, hd), jnp.float32),
            pltpu.VMEM((RB, hd), jnp.float32),
            pltpu.VMEM_SHARED((n_nodes, hd), jnp.float32),
        ],
    )
    def edge_pass(k_hbm, qv_hbm, ee_hbm, src_hbm, dst_hbm, out_hbm,
                  sidx, didx, kbuf, qvbuf, ebuf, zbuf, acc):
        cid = lax.axis_index("c")
        sid = lax.axis_index("s")
        wid = sid * NC + cid

        def zrow(r, carry):
            for j in range(nslice):
                zbuf[r, pl.ds(j * 16, 16)] = jnp.zeros((16,), jnp.float32)
            return carry

        lax.fori_loop(0, RB, zrow, 0)
        for t in range(n_rb):
            pltpu.sync_copy(zbuf, acc.at[pl.ds(sid * RT + t * RB, RB)])

        @pl.when(sid == 0)
        def _zero_tail():
            pltpu.sync_copy(zbuf.at[pl.ds(0, tail)], acc.at[pl.ds(NS * RT, tail)])

        plsc.subcore_barrier()

        def chunk_body(it, carry):
            ch = it * NW + wid

            @pl.when(ch < nchunks)
            def _():
                base = ch * C
                pltpu.sync_copy(src_hbm.at[pl.ds(base, C)], sidx)
                pltpu.sync_copy(dst_hbm.at[pl.ds(base, C)], didx)
                pltpu.sync_copy(k_hbm.at[didx], kbuf)
                pltpu.sync_copy(qv_hbm.at[sidx], qvbuf)
                pltpu.sync_copy(ee_hbm.at[pl.ds(base, C)], ebuf)

                @plsc.parallel_loop(0, C, unroll=4)
                def _row(r):
                    for j in range(nslice):
                        sl = pl.ds(j * 16, 16)
                        s = kbuf[r, sl] + qvbuf[r, sl] + ebuf[r, sl]
                        t = 1.0 + jnp.exp(-s)
                        ebuf[r, sl] = qvbuf[r, pl.ds(hd + j * 16, 16)] / t

                pltpu.sync_copy(ebuf, acc.at[didx], add=True)

            return carry

        lax.fori_loop(0, max_it, chunk_body, 0)
        plsc.subcore_barrier()

        for t in range(n_rb):
            r0 = sid * RT + t * RB
            pltpu.sync_copy(acc.at[pl.ds(r0, RB)], zbuf)
            pltpu.sync_copy(zbuf, out_hbm.at[pl.ds(cid * n_nodes + r0, RB)])

        @pl.when(sid == 0)
        def _copy_tail():
            pltpu.sync_copy(acc.at[pl.ds(NS * RT, tail)], zbuf.at[pl.ds(0, tail)])
            pltpu.sync_copy(zbuf.at[pl.ds(0, tail)],
                            out_hbm.at[pl.ds(cid * n_nodes + NS * RT, tail)])

    return edge_pass


def kernel(x, edge_index, edge_attr, Wn, bn_, We, be, Wk, bk, Wq, bq,
           Wv, bv, Ws, bs, Wed, bed, gamma, beta, Wh, bh):
    n_nodes = x.shape[0]
    n_edges = edge_index.shape[1]
    hd = Wn.shape[1]
    n_layers = Wk.shape[0]

    src = edge_index[0]
    dst = edge_index[1]

    h = _matmul(x, Wn, bn_, 2000)
    edge_pass = _make_edge_pass(n_nodes, n_edges, hd)

    for i in range(n_layers):
        ee = _ee(edge_attr, We, Wed[i], be, bed[i], 8000)
        k, qv, s = _qkvs(h, Wk[i], Wq[i], Wv[i], Ws[i],
                         bk[i], bq[i], bv[i], bs[i], 2000)
        agg2 = edge_pass(k, qv, ee, src, dst)
        h = _bn_update(agg2, s, h, gamma[i], beta[i])

    return _matmul(h, Wh, bh, 2000)


# speedup vs baseline: 3.3966x; 1.1546x over previous
import functools

import jax
import jax.numpy as jnp
from jax import lax
from jax.experimental import pallas as pl
from jax.experimental.pallas import tpu as pltpu
from jax.experimental.pallas import tpu_sc as plsc


def _mm_body(x_ref, w_ref, b_ref, o_ref):
    o_ref[...] = (
        jnp.dot(x_ref[...], w_ref[...], preferred_element_type=jnp.float32)
        + b_ref[...]
    )


def _matmul(x, w, b, bm):
    m, kdim = x.shape
    n = w.shape[1]
    return pl.pallas_call(
        _mm_body,
        grid=(m // bm,),
        in_specs=[
            pl.BlockSpec((bm, kdim), lambda i: (i, 0)),
            pl.BlockSpec((kdim, n), lambda i: (0, 0)),
            pl.BlockSpec((1, n), lambda i: (0, 0)),
        ],
        out_specs=pl.BlockSpec((bm, n), lambda i: (i, 0)),
        out_shape=jax.ShapeDtypeStruct((m, n), jnp.float32),
    )(x, w, b.reshape(1, n))


def _qkvs_body(h_ref, wk_ref, wq_ref, wv_ref, ws_ref,
               bk_ref, bq_ref, bv_ref, bs_ref,
               k_ref, qv_ref, s_ref):
    h = h_ref[...]
    hd = wk_ref.shape[1]
    k_ref[...] = jnp.dot(h, wk_ref[...], preferred_element_type=jnp.float32) + bk_ref[...]
    qv_ref[:, :hd] = jnp.dot(h, wq_ref[...], preferred_element_type=jnp.float32) + bq_ref[...]
    qv_ref[:, hd:] = jnp.dot(h, wv_ref[...], preferred_element_type=jnp.float32) + bv_ref[...]
    s_ref[...] = jnp.dot(h, ws_ref[...], preferred_element_type=jnp.float32) + bs_ref[...]


def _qkvs(h, wk, wq, wv, ws, bk, bq, bv, bs, bm):
    m, hd = h.shape
    wspec = pl.BlockSpec((hd, hd), lambda i: (0, 0))
    bspec = pl.BlockSpec((1, hd), lambda i: (0, 0))
    return pl.pallas_call(
        _qkvs_body,
        grid=(m // bm,),
        in_specs=[pl.BlockSpec((bm, hd), lambda i: (i, 0))]
        + [wspec] * 4 + [bspec] * 4,
        out_specs=[
            pl.BlockSpec((bm, hd), lambda i: (i, 0)),
            pl.BlockSpec((bm, 2 * hd), lambda i: (i, 0)),
            pl.BlockSpec((bm, hd), lambda i: (i, 0)),
        ],
        out_shape=[
            jax.ShapeDtypeStruct((m, hd), jnp.float32),
            jax.ShapeDtypeStruct((m, 2 * hd), jnp.float32),
            jax.ShapeDtypeStruct((m, hd), jnp.float32),
        ],
    )(h, wk, wq, wv, ws,
      bk.reshape(1, hd), bq.reshape(1, hd), bv.reshape(1, hd), bs.reshape(1, hd))


def _ee_body(ea_ref, we_ref, wed_ref, be_ref, bed_ref, o_ref):
    u = jnp.dot(we_ref[...], wed_ref[...], preferred_element_type=jnp.float32)
    c = jnp.dot(be_ref[...], wed_ref[...], preferred_element_type=jnp.float32) + bed_ref[...]
    o_ref[...] = jnp.dot(ea_ref[...], u, preferred_element_type=jnp.float32) + c


def _ee(edge_attr, we, wed_i, be, bed_i, be_blk):
    e_cnt, de = edge_attr.shape
    hd = we.shape[1]
    return pl.pallas_call(
        _ee_body,
        grid=(e_cnt // be_blk,),
        in_specs=[
            pl.BlockSpec((be_blk, de), lambda i: (i, 0)),
            pl.BlockSpec((de, hd), lambda i: (0, 0)),
            pl.BlockSpec((hd, hd), lambda i: (0, 0)),
            pl.BlockSpec((1, hd), lambda i: (0, 0)),
            pl.BlockSpec((1, hd), lambda i: (0, 0)),
        ],
        out_specs=pl.BlockSpec((be_blk, hd), lambda i: (i, 0)),
        out_shape=jax.ShapeDtypeStruct((e_cnt, hd), jnp.float32),
    )(edge_attr, we, wed_i, be.reshape(1, hd), bed_i.reshape(1, hd))


def _bn_update(agg2, s, h, gamma_i, beta_i):
    n_nodes, hd = h.shape

    def body(agg_ref, s_ref, h_ref, g_ref, b_ref, o_ref):
        a = agg_ref[...]
        nnew = a[:n_nodes] + a[n_nodes:] + s_ref[...]
        mean = jnp.mean(nnew, axis=0, keepdims=True)
        ctr = nnew - mean
        var = jnp.mean(ctr * ctr, axis=0, keepdims=True)
        nb = g_ref[...] * ctr * lax.rsqrt(var + 1e-5) + b_ref[...]
        o_ref[...] = (h_ref[...] + jnp.maximum(nb, 0.0)) * 0.5

    return pl.pallas_call(
        body,
        out_shape=jax.ShapeDtypeStruct((n_nodes, hd), jnp.float32),
    )(agg2, s, h, gamma_i.reshape(1, hd), beta_i.reshape(1, hd))


def _make_edge_pass(n_nodes, n_edges, hd):
    C = 32
    NC, NS = 2, 16
    NW = NC * NS
    SCN = 80
    NSC = 4
    rows_per_tile = NSC * SCN
    n_rows = NW * rows_per_tile
    RT = (n_nodes // NS) // 8 * 8
    RB = 48
    n_rb = RT // RB
    tail = n_nodes - NS * RT
    nslice = hd // 16
    n_acc = n_nodes + 8

    mesh = plsc.VectorSubcoreMesh(core_axis_name="c", subcore_axis_name="s")

    @functools.partial(
        pl.kernel,
        mesh=mesh,
        out_type=jax.ShapeDtypeStruct((NC * n_nodes, hd), jnp.float32),
        scratch_types=[
            pltpu.VMEM((SCN * C,), jnp.int32),
            pltpu.VMEM((SCN * C,), jnp.int32),
            pltpu.VMEM((2, C), jnp.int32),
            pltpu.VMEM((2, C, hd), jnp.float32),
            pltpu.VMEM((2, C, 2 * hd), jnp.float32),
            pltpu.VMEM((2, C, hd), jnp.float32),
            pltpu.VMEM((RB, hd), jnp.float32),
            pltpu.VMEM_SHARED((n_acc, hd), jnp.float32),
            pltpu.SemaphoreType.DMA,
            pltpu.SemaphoreType.DMA,
            pltpu.SemaphoreType.DMA,
        ],
    )
    def edge_pass(k_hbm, qv_hbm, ee_hbm, src_hbm, dst_hbm, out_hbm,
                  sidx, didx, dscat, kbuf, qvbuf, ebuf, zbuf, acc,
                  gsem, ssem0, ssem1):
        cid = lax.axis_index("c")
        sid = lax.axis_index("s")
        wid = sid * NC + cid
        row0 = wid * rows_per_tile

        def zrow(r, carry):
            for j in range(nslice):
                zbuf[r, pl.ds(j * 16, 16)] = jnp.zeros((16,), jnp.float32)
            return carry

        lax.fori_loop(0, RB, zrow, 0)
        for t in range(n_rb):
            pltpu.sync_copy(zbuf, acc.at[pl.ds(sid * RT + t * RB, RB)])

        @pl.when(sid == 0)
        def _zero_tail():
            pltpu.sync_copy(zbuf.at[pl.ds(0, tail)], acc.at[pl.ds(NS * RT, tail)])

        plsc.subcore_barrier()

        def fire_gathers(lcc, grow, s):
            bee = jnp.minimum(grow * C, n_edges - C)
            pltpu.async_copy(k_hbm.at[didx.at[pl.ds(lcc * C, C)]], kbuf.at[s], gsem)
            pltpu.async_copy(qv_hbm.at[sidx.at[pl.ds(lcc * C, C)]], qvbuf.at[s], gsem)
            pltpu.async_copy(ee_hbm.at[pl.ds(bee, C)], ebuf.at[s], gsem)

        def wait_gathers(s):
            pltpu.make_async_copy(k_hbm.at[didx.at[pl.ds(0, C)]], kbuf.at[s], gsem).wait()
            pltpu.make_async_copy(qv_hbm.at[sidx.at[pl.ds(0, C)]], qvbuf.at[s], gsem).wait()
            pltpu.make_async_copy(ee_hbm.at[pl.ds(0, C)], ebuf.at[s], gsem).wait()

        def fire_scatter(lcc, s, sem):
            for j in range(C // 16):
                dscat[s, pl.ds(j * 16, 16)] = didx[pl.ds(lcc * C + j * 16, 16)]
            pltpu.async_copy(ebuf.at[s], acc.at[dscat.at[s]], sem, add=True)

        def wait_scatter(s, sem):
            pltpu.make_async_copy(ebuf.at[s], acc.at[dscat.at[s]], sem).wait()

        def compute(s):
            @plsc.parallel_loop(0, C, unroll=4)
            def _row(r):
                for j in range(nslice):
                    sl = pl.ds(j * 16, 16)
                    g = kbuf[s, r, sl] + qvbuf[s, r, sl] + ebuf[s, r, sl]
                    t = 1.0 + jnp.exp(-g)
                    ebuf[s, r, sl] = qvbuf[s, r, pl.ds(hd + j * 16, 16)] / t

        def superchunk(sc, carry):
            scrow = row0 + sc * SCN

            @pl.when(sc > 0)
            def _drain():
                wait_scatter(0, ssem0)
                wait_scatter(1, ssem1)

            pltpu.sync_copy(src_hbm.at[pl.ds(scrow * C, SCN * C)], sidx)
            pltpu.sync_copy(dst_hbm.at[pl.ds(scrow * C, SCN * C)], didx)
            fire_gathers(0, scrow, 0)

            def pair(cc2, carry2):
                c0 = 2 * cc2
                c1 = c0 + 1
                wait_gathers(0)

                @pl.when(cc2 > 0)
                def _ws1():
                    wait_scatter(1, ssem1)

                fire_gathers(c1, scrow + c1, 1)
                compute(0)
                fire_scatter(c0, 0, ssem0)
                wait_gathers(1)

                @pl.when(cc2 < SCN // 2 - 1)
                def _next0():
                    wait_scatter(0, ssem0)
                    fire_gathers(c1 + 1, scrow + c1 + 1, 0)

                compute(1)
                fire_scatter(c1, 1, ssem1)
                return carry2

            lax.fori_loop(0, SCN // 2, pair, 0)
            return carry

        lax.fori_loop(0, NSC, superchunk, 0)
        wait_scatter(0, ssem0)
        wait_scatter(1, ssem1)
        plsc.subcore_barrier()

        for t in range(n_rb):
            r0 = sid * RT + t * RB
            pltpu.sync_copy(acc.at[pl.ds(r0, RB)], zbuf)
            pltpu.sync_copy(zbuf, out_hbm.at[pl.ds(cid * n_nodes + r0, RB)])

        @pl.when(sid == 0)
        def _copy_tail():
            pltpu.sync_copy(acc.at[pl.ds(NS * RT, tail)], zbuf.at[pl.ds(0, tail)])
            pltpu.sync_copy(zbuf.at[pl.ds(0, tail)],
                            out_hbm.at[pl.ds(cid * n_nodes + NS * RT, tail)])

    pad_n = n_rows * C - n_edges

    def run(k, qv, ee, src, dst):
        srcp = jnp.concatenate([src, jnp.zeros((pad_n,), jnp.int32)])
        dstp = jnp.concatenate([dst, jnp.full((pad_n,), n_nodes, jnp.int32)])
        return edge_pass(k, qv, ee, srcp, dstp)

    return run


def kernel(x, edge_index, edge_attr, Wn, bn_, We, be, Wk, bk, Wq, bq,
           Wv, bv, Ws, bs, Wed, bed, gamma, beta, Wh, bh):
    n_nodes = x.shape[0]
    n_edges = edge_index.shape[1]
    hd = Wn.shape[1]
    n_layers = Wk.shape[0]

    src = edge_index[0]
    dst = edge_index[1]

    h = _matmul(x, Wn, bn_, 2000)
    edge_pass = _make_edge_pass(n_nodes, n_edges, hd)

    for i in range(n_layers):
        ee = _ee(edge_attr, We, Wed[i], be, bed[i], 8000)
        k, qv, s = _qkvs(h, Wk[i], Wq[i], Wv[i], Ws[i],
                         bk[i], bq[i], bv[i], bs[i], 2000)
        agg2 = edge_pass(k, qv, ee, src, dst)
        h = _bn_update(agg2, s, h, gamma[i], beta[i])

    return _matmul(h, Wh, bh, 2000)


# speedup vs baseline: 3.5288x; 1.0389x over previous
import functools

import jax
import jax.numpy as jnp
from jax import lax
from jax.experimental import pallas as pl
from jax.experimental.pallas import tpu as pltpu
from jax.experimental.pallas import tpu_sc as plsc


def _mm_body(x_ref, w_ref, b_ref, o_ref):
    o_ref[...] = (
        jnp.dot(x_ref[...], w_ref[...], preferred_element_type=jnp.float32)
        + b_ref[...]
    )


def _matmul(x, w, b, bm):
    m, kdim = x.shape
    n = w.shape[1]
    return pl.pallas_call(
        _mm_body,
        grid=(m // bm,),
        in_specs=[
            pl.BlockSpec((bm, kdim), lambda i: (i, 0)),
            pl.BlockSpec((kdim, n), lambda i: (0, 0)),
            pl.BlockSpec((1, n), lambda i: (0, 0)),
        ],
        out_specs=pl.BlockSpec((bm, n), lambda i: (i, 0)),
        out_shape=jax.ShapeDtypeStruct((m, n), jnp.float32),
    )(x, w, b.reshape(1, n))


def _qkvs_body(h_ref, wk_ref, wq_ref, wv_ref, ws_ref,
               bk_ref, bq_ref, bv_ref, bs_ref,
               k_ref, qv_ref, s_ref):
    h = h_ref[...]
    hd = wk_ref.shape[1]
    k_ref[...] = jnp.dot(h, wk_ref[...], preferred_element_type=jnp.float32) + bk_ref[...]
    qv_ref[:, :hd] = jnp.dot(h, wq_ref[...], preferred_element_type=jnp.float32) + bq_ref[...]
    qv_ref[:, hd:] = jnp.dot(h, wv_ref[...], preferred_element_type=jnp.float32) + bv_ref[...]
    s_ref[...] = jnp.dot(h, ws_ref[...], preferred_element_type=jnp.float32) + bs_ref[...]


def _qkvs(h, wk, wq, wv, ws, bk, bq, bv, bs, bm):
    m, hd = h.shape
    wspec = pl.BlockSpec((hd, hd), lambda i: (0, 0))
    bspec = pl.BlockSpec((1, hd), lambda i: (0, 0))
    return pl.pallas_call(
        _qkvs_body,
        grid=(m // bm,),
        in_specs=[pl.BlockSpec((bm, hd), lambda i: (i, 0))]
        + [wspec] * 4 + [bspec] * 4,
        out_specs=[
            pl.BlockSpec((bm, hd), lambda i: (i, 0)),
            pl.BlockSpec((bm, 2 * hd), lambda i: (i, 0)),
            pl.BlockSpec((bm, hd), lambda i: (i, 0)),
        ],
        out_shape=[
            jax.ShapeDtypeStruct((m, hd), jnp.float32),
            jax.ShapeDtypeStruct((m, 2 * hd), jnp.float32),
            jax.ShapeDtypeStruct((m, hd), jnp.float32),
        ],
    )(h, wk, wq, wv, ws,
      bk.reshape(1, hd), bq.reshape(1, hd), bv.reshape(1, hd), bs.reshape(1, hd))


def _ee_body(ea_ref, we_ref, wed_ref, be_ref, bed_ref, o_ref):
    u = jnp.dot(we_ref[...], wed_ref[...], preferred_element_type=jnp.float32)
    c = jnp.dot(be_ref[...], wed_ref[...], preferred_element_type=jnp.float32) + bed_ref[...]
    o_ref[...] = jnp.dot(ea_ref[...], u, preferred_element_type=jnp.float32) + c


def _ee(edge_attr, we, wed_i, be, bed_i, be_blk):
    e_cnt, de = edge_attr.shape
    hd = we.shape[1]
    return pl.pallas_call(
        _ee_body,
        grid=(e_cnt // be_blk,),
        in_specs=[
            pl.BlockSpec((be_blk, de), lambda i: (i, 0)),
            pl.BlockSpec((de, hd), lambda i: (0, 0)),
            pl.BlockSpec((hd, hd), lambda i: (0, 0)),
            pl.BlockSpec((1, hd), lambda i: (0, 0)),
            pl.BlockSpec((1, hd), lambda i: (0, 0)),
        ],
        out_specs=pl.BlockSpec((be_blk, hd), lambda i: (i, 0)),
        out_shape=jax.ShapeDtypeStruct((e_cnt, hd), jnp.float32),
    )(edge_attr, we, wed_i, be.reshape(1, hd), bed_i.reshape(1, hd))


def _bn_update(agg2, s, h, gamma_i, beta_i):
    n_nodes, hd = h.shape

    def body(agg_ref, s_ref, h_ref, g_ref, b_ref, o_ref):
        a = agg_ref[...]
        nnew = a[:n_nodes] + a[n_nodes:] + s_ref[...]
        mean = jnp.mean(nnew, axis=0, keepdims=True)
        ctr = nnew - mean
        var = jnp.mean(ctr * ctr, axis=0, keepdims=True)
        nb = g_ref[...] * ctr * lax.rsqrt(var + 1e-5) + b_ref[...]
        o_ref[...] = (h_ref[...] + jnp.maximum(nb, 0.0)) * 0.5

    return pl.pallas_call(
        body,
        out_shape=jax.ShapeDtypeStruct((n_nodes, hd), jnp.float32),
    )(agg2, s, h, gamma_i.reshape(1, hd), beta_i.reshape(1, hd))


def _make_edge_pass(n_nodes, n_edges, hd):
    C = 40
    NC, NS = 2, 16
    NW = NC * NS
    SCN = 64
    NSC = 4
    rows_per_tile = NSC * SCN
    n_rows = NW * rows_per_tile
    RT = (n_nodes // NS) // 8 * 8
    RB = 24
    n_rb = RT // RB
    tail = n_nodes - NS * RT
    nslice = hd // 16
    n_acc = n_nodes + 8

    mesh = plsc.VectorSubcoreMesh(core_axis_name="c", subcore_axis_name="s")

    @functools.partial(
        pl.kernel,
        mesh=mesh,
        out_type=jax.ShapeDtypeStruct((NC * n_nodes, hd), jnp.float32),
        scratch_types=[
            pltpu.VMEM((SCN * C,), jnp.int32),
            pltpu.VMEM((SCN * C,), jnp.int32),
            pltpu.VMEM((2, C), jnp.int32),
            pltpu.VMEM((2, C, hd), jnp.float32),
            pltpu.VMEM((2, C, 2 * hd), jnp.float32),
            pltpu.VMEM((2, C, hd), jnp.float32),
            pltpu.VMEM_SHARED((n_acc, hd), jnp.float32),
            pltpu.SemaphoreType.DMA,
            pltpu.SemaphoreType.DMA,
            pltpu.SemaphoreType.DMA,
        ],
    )
    def edge_pass(k_hbm, qv_hbm, ee_hbm, src_hbm, dst_hbm, out_hbm,
                  sidx, didx, dscat, kbuf, qvbuf, ebuf, acc,
                  gsem, ssem0, ssem1):
        cid = lax.axis_index("c")
        sid = lax.axis_index("s")
        wid = sid * NC + cid
        row0 = wid * rows_per_tile
        zbuf = ebuf.at[0, pl.ds(0, RB)]

        def zrow(r, carry):
            for j in range(nslice):
                ebuf[0, r, pl.ds(j * 16, 16)] = jnp.zeros((16,), jnp.float32)
            return carry

        lax.fori_loop(0, RB, zrow, 0)
        for t in range(n_rb):
            pltpu.async_copy(zbuf, acc.at[pl.ds(sid * RT + t * RB, RB)], gsem)

        @pl.when(sid == 0)
        def _zero_tail():
            pltpu.async_copy(ebuf.at[0, pl.ds(0, tail)],
                             acc.at[pl.ds(NS * RT, tail)], gsem)

        for t in range(n_rb):
            pltpu.make_async_copy(zbuf, acc.at[pl.ds(sid * RT + t * RB, RB)],
                                  gsem).wait()

        @pl.when(sid == 0)
        def _zero_tail_wait():
            pltpu.make_async_copy(ebuf.at[0, pl.ds(0, tail)],
                                  acc.at[pl.ds(NS * RT, tail)], gsem).wait()

        plsc.subcore_barrier()

        def fire_gathers(lcc, grow, s):
            bee = jnp.minimum(grow * C, n_edges - C)
            pltpu.async_copy(k_hbm.at[didx.at[pl.ds(lcc * C, C)]], kbuf.at[s], gsem)
            pltpu.async_copy(qv_hbm.at[sidx.at[pl.ds(lcc * C, C)]], qvbuf.at[s], gsem)
            pltpu.async_copy(ee_hbm.at[pl.ds(bee, C)], ebuf.at[s], gsem)

        def wait_gathers(s):
            pltpu.make_async_copy(k_hbm.at[didx.at[pl.ds(0, C)]], kbuf.at[s], gsem).wait()
            pltpu.make_async_copy(qv_hbm.at[sidx.at[pl.ds(0, C)]], qvbuf.at[s], gsem).wait()
            pltpu.make_async_copy(ee_hbm.at[pl.ds(0, C)], ebuf.at[s], gsem).wait()

        def fire_scatter(lcc, s, sem):
            offs = sorted({min(j * 16, C - 16) for j in range((C + 15) // 16)})
            for o in offs:
                dscat[s, pl.ds(o, 16)] = didx[pl.ds(lcc * C + o, 16)]
            pltpu.async_copy(ebuf.at[s], acc.at[dscat.at[s]], sem, add=True)

        def wait_scatter(s, sem):
            pltpu.make_async_copy(ebuf.at[s], acc.at[dscat.at[s]], sem).wait()

        def compute(s):
            @plsc.parallel_loop(0, C, unroll=4)
            def _row(r):
                for j in range(nslice):
                    sl = pl.ds(j * 16, 16)
                    g = kbuf[s, r, sl] + qvbuf[s, r, sl] + ebuf[s, r, sl]
                    t = 1.0 + jnp.exp(-g)
                    ebuf[s, r, sl] = qvbuf[s, r, pl.ds(hd + j * 16, 16)] / t

        def superchunk(sc, carry):
            scrow = row0 + sc * SCN

            @pl.when(sc > 0)
            def _drain():
                wait_scatter(0, ssem0)
                wait_scatter(1, ssem1)

            pltpu.sync_copy(src_hbm.at[pl.ds(scrow * C, SCN * C)], sidx)
            pltpu.sync_copy(dst_hbm.at[pl.ds(scrow * C, SCN * C)], didx)
            fire_gathers(0, scrow, 0)

            def pair(cc2, carry2):
                c0 = 2 * cc2
                c1 = c0 + 1
                wait_gathers(0)

                @pl.when(cc2 > 0)
                def _ws1():
                    wait_scatter(1, ssem1)

                fire_gathers(c1, scrow + c1, 1)
                compute(0)
                fire_scatter(c0, 0, ssem0)
                wait_gathers(1)

                @pl.when(cc2 < SCN // 2 - 1)
                def _next0():
                    wait_scatter(0, ssem0)
                    fire_gathers(c1 + 1, scrow + c1 + 1, 0)

                compute(1)
                fire_scatter(c1, 1, ssem1)
                return carry2

            lax.fori_loop(0, SCN // 2, pair, 0)
            return carry

        lax.fori_loop(0, NSC, superchunk, 0)
        wait_scatter(0, ssem0)
        wait_scatter(1, ssem1)
        plsc.subcore_barrier()

        sems = (ssem0, ssem1)
        for t in range(n_rb):
            r0 = sid * RT + t * RB
            st = ebuf.at[t % 2, pl.ds(0, RB)]
            ohb = out_hbm.at[pl.ds(cid * n_nodes + r0, RB)]
            if t >= 2:
                p0 = sid * RT + (t - 2) * RB
                pltpu.make_async_copy(
                    ebuf.at[t % 2, pl.ds(0, RB)],
                    out_hbm.at[pl.ds(cid * n_nodes + p0, RB)],
                    sems[t % 2]).wait()
            pltpu.sync_copy(acc.at[pl.ds(r0, RB)], st)
            pltpu.async_copy(st, ohb, sems[t % 2])
        for t in (n_rb - 2, n_rb - 1):
            r0 = sid * RT + t * RB
            pltpu.make_async_copy(
                ebuf.at[t % 2, pl.ds(0, RB)],
                out_hbm.at[pl.ds(cid * n_nodes + r0, RB)],
                sems[t % 2]).wait()

        @pl.when(sid == 0)
        def _copy_tail():
            pltpu.sync_copy(acc.at[pl.ds(NS * RT, tail)],
                            ebuf.at[0, pl.ds(0, tail)])
            pltpu.sync_copy(ebuf.at[0, pl.ds(0, tail)],
                            out_hbm.at[pl.ds(cid * n_nodes + NS * RT, tail)])

    pad_n = n_rows * C - n_edges

    def run(k, qv, ee, src, dst):
        srcp = jnp.concatenate([src, jnp.zeros((pad_n,), jnp.int32)])
        dstp = jnp.concatenate([dst, jnp.full((pad_n,), n_nodes, jnp.int32)])
        return edge_pass(k, qv, ee, srcp, dstp)

    return run


def kernel(x, edge_index, edge_attr, Wn, bn_, We, be, Wk, bk, Wq, bq,
           Wv, bv, Ws, bs, Wed, bed, gamma, beta, Wh, bh):
    n_nodes = x.shape[0]
    n_edges = edge_index.shape[1]
    hd = Wn.shape[1]
    n_layers = Wk.shape[0]

    src = edge_index[0]
    dst = edge_index[1]

    h = _matmul(x, Wn, bn_, 2000)
    edge_pass = _make_edge_pass(n_nodes, n_edges, hd)

    for i in range(n_layers):
        ee = _ee(edge_attr, We, Wed[i], be, bed[i], 8000)
        k, qv, s = _qkvs(h, Wk[i], Wq[i], Wv[i], Ws[i],
                         bk[i], bq[i], bv[i], bs[i], 2000)
        agg2 = edge_pass(k, qv, ee, src, dst)
        h = _bn_update(agg2, s, h, gamma[i], beta[i])

    return _matmul(h, Wh, bh, 2000)
